# baseline (speedup 1.0000x reference)
import functools

import jax
import jax.numpy as jnp
from jax import lax
from jax.experimental import pallas as pl
from jax.experimental.pallas import tpu as pltpu
from jax.experimental.pallas import tpu_sc as plsc

_NC = 2
_NS = 16
_NW = _NC * _NS
_SB = 128


def _sc_pool(idx_flat, mask_flat, emb_pad, B, L):
    VF = emb_pad.shape[0]
    ED = 16
    EDP = ED + 1
    V = VF // EDP
    rows_per_w = B // _NW
    n_super = rows_per_w // _SB
    groups = _SB // 16

    mesh = plsc.VectorSubcoreMesh(core_axis_name="c", subcore_axis_name="s")

    @functools.partial(
        pl.kernel,
        out_type=jax.ShapeDtypeStruct((EDP, B), jnp.float32),
        mesh=mesh,
        compiler_params=pltpu.CompilerParams(use_tc_tiling_on_sc=False,
                                             needs_layout_passes=False),
        scratch_types=[
            pltpu.VMEM((V * EDP,), jnp.float32),
            pltpu.VMEM((2, _SB * L), jnp.int32),
            pltpu.VMEM((2, _SB * L), jnp.float32),
            pltpu.VMEM((2, EDP, _SB), jnp.float32),
            pltpu.SemaphoreType.DMA,
            pltpu.SemaphoreType.DMA,
            pltpu.SemaphoreType.DMA,
            pltpu.SemaphoreType.DMA,
        ],
    )
    def pool_k(idx_hbm, mask_hbm, table_hbm, pooled_hbm,
               table_v, idx_v, mask_v, pbl_v, in_sem0, in_sem1,
               out_sem0, out_sem1):
        wid = lax.axis_index("s") * _NC + lax.axis_index("c")
        base = wid * rows_per_w
        pltpu.sync_copy(table_hbm, table_v)
        lanes = lax.iota(jnp.int32, 16)
        in_sems = (in_sem0, in_sem1)
        out_sems = (out_sem0, out_sem1)

        def stage(sb, slot):
            elbase = (base + sb * _SB) * L
            a = pltpu.async_copy(idx_hbm.at[pl.ds(elbase, _SB * L)],
                                 idx_v.at[slot], in_sems[slot])
            b = pltpu.async_copy(mask_hbm.at[pl.ds(elbase, _SB * L)],
                                 mask_v.at[slot], in_sems[slot])
            return a, b

        pending_in = {0: stage(0, 0)}
        pending_out = {}

        for sb in range(n_super):
            slot = sb % 2
            for h in pending_in.pop(sb):
                h.wait()
            if sb + 1 < n_super:
                pending_in[sb + 1] = stage(sb + 1, 1 - slot)
            for h in pending_out.pop(sb - 2, ()):
                h.wait()

            for g in range(groups):
                r0 = g * 16
                rl = (lanes + r0) * L

                def body_l(l, accs, _slot=slot, _rl=rl):
                    lful = _rl + l
                    idxc = plsc.load_gather(idx_v.at[_slot], [lful])
                    mc = plsc.load_gather(mask_v.at[_slot], [lful])
                    flat = idxc * EDP
                    flat_r = [flat + r for r in range(8)]
                    new = []
                    for d in range(ED):
                        q, r = divmod(d, 8)
                        gth = plsc.load_gather(
                            table_v.at[pl.ds(8 * q, V * EDP - ED)],
                            [flat_r[r]])
                        new.append(accs[d] + gth * mc)
                    new.append(accs[ED] + mc)
                    return tuple(new)

                zero = jnp.zeros((16,), jnp.float32)
                accs = lax.fori_loop(0, L, body_l, (zero,) * (ED + 1))
                for d in range(EDP):
                    pbl_v[slot, d, pl.ds(r0, 16)] = accs[d]

            rowbase = base + sb * _SB
            pending_out[sb] = (
                pltpu.async_copy(pbl_v.at[slot],
                                 pooled_hbm.at[:, pl.ds(rowbase, _SB)],
                                 out_sems[slot]),
            )
        for hs in pending_out.values():
            for h in hs:
                h.wait()

    return pool_k(idx_flat, mask_flat, emb_pad)


def _mlp(pooled_t, coverage_t, w1e, w1f, w1c, b1r, W2, b2r):
    EDP, B = pooled_t.shape
    ED = EDP - 1
    H = w1e.shape[1]
    O = W2.shape[1]
    BLK = 4096
    grid = (B // BLK,)
    L = 200.0

    def body(pooled_ref, cov_ref, w1e_ref, w1f_ref, w1c_ref,
             b1_ref, w2_ref, b2_ref, out_ref):
        ms = pooled_ref[ED:EDP, :]
        pd = pooled_ref[0:ED, :] / jnp.maximum(ms, 1.0)
        frac = jnp.transpose(ms) * (1.0 / L)
        cov = jnp.transpose(cov_ref[...])
        h = lax.dot_general(pd, w1e_ref[...], (((0,), (0,)), ((), ())),
                            preferred_element_type=jnp.float32)
        h = h + frac * w1f_ref[...] + cov * w1c_ref[...] + b1_ref[...]
        h = h / (1.0 + jnp.exp(-h))
        out_ref[...] = jnp.dot(h, w2_ref[...],
                               preferred_element_type=jnp.float32) + b2_ref[...]

    rep = lambda i: (0, 0)
    return pl.pallas_call(
        body,
        grid=grid,
        in_specs=[
            pl.BlockSpec((EDP, BLK), lambda i: (0, i)),
            pl.BlockSpec((1, BLK), lambda i: (0, i)),
            pl.BlockSpec((ED, H), rep),
            pl.BlockSpec((1, H), rep),
            pl.BlockSpec((1, H), rep),
            pl.BlockSpec((1, H), rep),
            pl.BlockSpec((H, O), rep),
            pl.BlockSpec((1, O), rep),
        ],
        out_specs=pl.BlockSpec((BLK, O), lambda i: (i, 0)),
        out_shape=jax.ShapeDtypeStruct((B, O), jnp.float32),
    )(pooled_t, coverage_t, w1e, w1f, w1c, b1r, W2, b2r)


def kernel(input_idx, obs_mask, coverage, emb_table, W1, b1, W2, b2):
    B, L = input_idx.shape
    ED = emb_table.shape[1]
    emb_pad = jnp.pad(emb_table, ((0, 0), (0, 1))).reshape(-1)
    pooled_t = _sc_pool(input_idx.reshape(-1), obs_mask.reshape(-1),
                        emb_pad, B, L)
    w1e = W1[:ED]
    w1f = W1[ED:ED + 1]
    w1c = W1[ED + 1:ED + 2]
    return _mlp(pooled_t, coverage.reshape(1, B),
                w1e, w1f, w1c, b1.reshape(1, -1), W2, b2.reshape(1, -1))


# speedup vs baseline: 1.0015x; 1.0015x over previous
import functools

import jax
import jax.numpy as jnp
from jax import lax
from jax.experimental import pallas as pl
from jax.experimental.pallas import tpu as pltpu
from jax.experimental.pallas import tpu_sc as plsc

_NC = 2
_NS = 16
_NW = _NC * _NS
_SB = 64


def _sc_pool(idx_flat, mask_flat, emb_pad, B, L):
    VF = emb_pad.shape[0]
    ED = 16
    EDP = ED + 1
    V = VF // EDP
    rows_per_w = B // _NW
    n_super = rows_per_w // _SB
    groups = _SB // 16

    mesh = plsc.VectorSubcoreMesh(core_axis_name="c", subcore_axis_name="s")

    @functools.partial(
        pl.kernel,
        out_type=jax.ShapeDtypeStruct((EDP, B), jnp.float32),
        mesh=mesh,
        compiler_params=pltpu.CompilerParams(use_tc_tiling_on_sc=False,
                                             needs_layout_passes=False),
        scratch_types=[
            pltpu.VMEM((V * EDP,), jnp.float32),
            pltpu.VMEM((2, _SB * L), jnp.int32),
            pltpu.VMEM((2, _SB * L), jnp.float32),
            pltpu.VMEM((2, EDP, _SB), jnp.float32),
            pltpu.SemaphoreType.DMA,
            pltpu.SemaphoreType.DMA,
            pltpu.SemaphoreType.DMA,
            pltpu.SemaphoreType.DMA,
        ],
    )
    def pool_k(idx_hbm, mask_hbm, table_hbm, pooled_hbm,
               table_v, idx_v, mask_v, pbl_v, in_sem0, in_sem1,
               out_sem0, out_sem1):
        wid = lax.axis_index("s") * _NC + lax.axis_index("c")
        base = wid * rows_per_w
        pltpu.sync_copy(table_hbm, table_v)
        lanes = lax.iota(jnp.int32, 16)
        in_sems = (in_sem0, in_sem1)
        out_sems = (out_sem0, out_sem1)

        def stage(sb, slot):
            elbase = (base + sb * _SB) * L
            a = pltpu.async_copy(idx_hbm.at[pl.ds(elbase, _SB * L)],
                                 idx_v.at[slot], in_sems[slot])
            b = pltpu.async_copy(mask_hbm.at[pl.ds(elbase, _SB * L)],
                                 mask_v.at[slot], in_sems[slot])
            return a, b

        pending_in = {0: stage(0, 0)}
        pending_out = {}

        for sb in range(n_super):
            slot = sb % 2
            for h in pending_in.pop(sb):
                h.wait()
            if sb + 1 < n_super:
                pending_in[sb + 1] = stage(sb + 1, 1 - slot)
            for h in pending_out.pop(sb - 2, ()):
                h.wait()

            for g in range(groups):
                r0 = g * 16
                rl = (lanes + r0) * L

                def body_l(l, accs, _slot=slot, _rl=rl):
                    lful = _rl + l
                    idxc = plsc.load_gather(idx_v.at[_slot], [lful])
                    mc = plsc.load_gather(mask_v.at[_slot], [lful])
                    flat = idxc * EDP
                    flat_r = [flat + r for r in range(8)]
                    new = []
                    for d in range(ED):
                        q, r = divmod(d, 8)
                        gth = plsc.load_gather(
                            table_v.at[pl.ds(8 * q, V * EDP - ED)],
                            [flat_r[r]])
                        new.append(accs[d] + gth * mc)
                    new.append(accs[ED] + mc)
                    return tuple(new)

                zero = jnp.zeros((16,), jnp.float32)
                accs = lax.fori_loop(0, L, body_l, (zero,) * (ED + 1))
                for d in range(EDP):
                    pbl_v[slot, d, pl.ds(r0, 16)] = accs[d]

            rowbase = base + sb * _SB
            pending_out[sb] = (
                pltpu.async_copy(pbl_v.at[slot],
                                 pooled_hbm.at[:, pl.ds(rowbase, _SB)],
                                 out_sems[slot]),
            )
        for hs in pending_out.values():
            for h in hs:
                h.wait()

    return pool_k(idx_flat, mask_flat, emb_pad)


def _mlp(pooled_t, coverage_t, w1e, w1f, w1c, b1r, W2, b2r):
    EDP, B = pooled_t.shape
    ED = EDP - 1
    H = w1e.shape[1]
    O = W2.shape[1]
    BLK = 2048
    grid = (B // BLK,)
    L = 200.0

    def body(pooled_ref, cov_ref, w1e_ref, w1f_ref, w1c_ref,
             b1_ref, w2_ref, b2_ref, out_ref):
        ms = pooled_ref[ED:EDP, :]
        pd = pooled_ref[0:ED, :] / jnp.maximum(ms, 1.0)
        frac = jnp.transpose(ms) * (1.0 / L)
        cov = jnp.transpose(cov_ref[...])
        h = lax.dot_general(pd, w1e_ref[...], (((0,), (0,)), ((), ())),
                            preferred_element_type=jnp.float32)
        h = h + frac * w1f_ref[...] + cov * w1c_ref[...] + b1_ref[...]
        h = h / (1.0 + jnp.exp(-h))
        out_ref[...] = jnp.dot(h, w2_ref[...],
                               preferred_element_type=jnp.float32) + b2_ref[...]

    rep = lambda i: (0, 0)
    return pl.pallas_call(
        body,
        grid=grid,
        in_specs=[
            pl.BlockSpec((EDP, BLK), lambda i: (0, i)),
            pl.BlockSpec((1, BLK), lambda i: (0, i)),
            pl.BlockSpec((ED, H), rep),
            pl.BlockSpec((1, H), rep),
            pl.BlockSpec((1, H), rep),
            pl.BlockSpec((1, H), rep),
            pl.BlockSpec((H, O), rep),
            pl.BlockSpec((1, O), rep),
        ],
        out_specs=pl.BlockSpec((BLK, O), lambda i: (i, 0)),
        out_shape=jax.ShapeDtypeStruct((B, O), jnp.float32),
    )(pooled_t, coverage_t, w1e, w1f, w1c, b1r, W2, b2r)


def kernel(input_idx, obs_mask, coverage, emb_table, W1, b1, W2, b2):
    B, L = input_idx.shape
    ED = emb_table.shape[1]
    emb_pad = jnp.pad(emb_table, ((0, 0), (0, 1))).reshape(-1)
    pooled_t = _sc_pool(input_idx.reshape(-1), obs_mask.reshape(-1),
                        emb_pad, B, L)
    w1e = W1[:ED]
    w1f = W1[ED:ED + 1]
    w1c = W1[ED + 1:ED + 2]
    return _mlp(pooled_t, coverage.reshape(1, B),
                w1e, w1f, w1c, b1.reshape(1, -1), W2, b2.reshape(1, -1))


# speedup vs baseline: 1.0057x; 1.0042x over previous
import functools

import jax
import jax.numpy as jnp
from jax import lax
from jax.experimental import pallas as pl
from jax.experimental.pallas import tpu as pltpu
from jax.experimental.pallas import tpu_sc as plsc

_NC = 2
_NS = 16
_NW = _NC * _NS
_SB = 64


def _sc_pool(idx_flat, mask_flat, emb_pad, B, L):
    VF = emb_pad.shape[0]
    ED = 16
    EDP = ED + 1
    V = VF // EDP
    rows_per_w = B // _NW
    n_super = rows_per_w // _SB
    groups = _SB // 16

    mesh = plsc.VectorSubcoreMesh(core_axis_name="c", subcore_axis_name="s")

    @functools.partial(
        pl.kernel,
        out_type=jax.ShapeDtypeStruct((EDP, B), jnp.float32),
        mesh=mesh,
        compiler_params=pltpu.CompilerParams(use_tc_tiling_on_sc=False,
                                             needs_layout_passes=False),
        scratch_types=[
            pltpu.VMEM((V * EDP,), jnp.float32),
            pltpu.VMEM((2, _SB * L), jnp.int32),
            pltpu.VMEM((2, _SB * L), jnp.float32),
            pltpu.VMEM((2, EDP, _SB), jnp.float32),
            pltpu.SemaphoreType.DMA,
            pltpu.SemaphoreType.DMA,
            pltpu.SemaphoreType.DMA,
            pltpu.SemaphoreType.DMA,
        ],
    )
    def pool_k(idx_hbm, mask_hbm, table_hbm, pooled_hbm,
               table_v, idx_v, mask_v, pbl_v, in_sem0, in_sem1,
               out_sem0, out_sem1):
        wid = lax.axis_index("s") * _NC + lax.axis_index("c")
        base = wid * rows_per_w
        lanes = lax.iota(jnp.int32, 16)
        in_sems = (in_sem0, in_sem1)
        out_sems = (out_sem0, out_sem1)

        def stage(sb, slot):
            elbase = (base + sb * _SB) * L
            a = pltpu.async_copy(idx_hbm.at[pl.ds(elbase, _SB * L)],
                                 idx_v.at[slot], in_sems[slot])
            b = pltpu.async_copy(mask_hbm.at[pl.ds(elbase, _SB * L)],
                                 mask_v.at[slot], in_sems[slot])
            return a, b

        pending_in = {0: stage(0, 0)}
        pltpu.sync_copy(table_hbm, table_v)
        pending_out = {}

        for sb in range(n_super):
            slot = sb % 2
            for h in pending_in.pop(sb):
                h.wait()
            if sb + 1 < n_super:
                pending_in[sb + 1] = stage(sb + 1, 1 - slot)
            for h in pending_out.pop(sb - 2, ()):
                h.wait()

            for g in range(groups):
                r0 = g * 16
                rl = (lanes + r0) * L

                def body_l(l, accs, _slot=slot, _rl=rl):
                    lful = _rl + l
                    idxc = plsc.load_gather(idx_v.at[_slot], [lful])
                    mc = plsc.load_gather(mask_v.at[_slot], [lful])
                    flat = idxc * EDP
                    flat_r = [flat + r for r in range(8)]
                    new = []
                    for d in range(ED):
                        q, r = divmod(d, 8)
                        gth = plsc.load_gather(
                            table_v.at[pl.ds(8 * q, V * EDP - ED)],
                            [flat_r[r]])
                        new.append(accs[d] + gth * mc)
                    new.append(accs[ED] + mc)
                    return tuple(new)

                zero = jnp.zeros((16,), jnp.float32)
                accs = lax.fori_loop(0, L, body_l, (zero,) * (ED + 1))
                for d in range(EDP):
                    pbl_v[slot, d, pl.ds(r0, 16)] = accs[d]

            rowbase = base + sb * _SB
            pending_out[sb] = (
                pltpu.async_copy(pbl_v.at[slot],
                                 pooled_hbm.at[:, pl.ds(rowbase, _SB)],
                                 out_sems[slot]),
            )
        for hs in pending_out.values():
            for h in hs:
                h.wait()

    return pool_k(idx_flat, mask_flat, emb_pad)


def _mlp(pooled_t, coverage_t, w1e, w1f, w1c, b1r, W2, b2r):
    EDP, B = pooled_t.shape
    ED = EDP - 1
    H = w1e.shape[1]
    O = W2.shape[1]
    BLK = 2048
    grid = (B // BLK,)
    L = 200.0

    def body(pooled_ref, cov_ref, w1e_ref, w1f_ref, w1c_ref,
             b1_ref, w2_ref, b2_ref, out_ref):
        ms = pooled_ref[ED:EDP, :]
        pd = pooled_ref[0:ED, :] / jnp.maximum(ms, 1.0)
        frac = jnp.transpose(ms) * (1.0 / L)
        cov = jnp.transpose(cov_ref[...])
        h = lax.dot_general(pd, w1e_ref[...], (((0,), (0,)), ((), ())),
                            preferred_element_type=jnp.float32)
        h = h + frac * w1f_ref[...] + cov * w1c_ref[...] + b1_ref[...]
        h = h / (1.0 + jnp.exp(-h))
        out_ref[...] = jnp.dot(h, w2_ref[...],
                               preferred_element_type=jnp.float32) + b2_ref[...]

    rep = lambda i: (0, 0)
    return pl.pallas_call(
        body,
        grid=grid,
        in_specs=[
            pl.BlockSpec((EDP, BLK), lambda i: (0, i)),
            pl.BlockSpec((1, BLK), lambda i: (0, i)),
            pl.BlockSpec((ED, H), rep),
            pl.BlockSpec((1, H), rep),
            pl.BlockSpec((1, H), rep),
            pl.BlockSpec((1, H), rep),
            pl.BlockSpec((H, O), rep),
            pl.BlockSpec((1, O), rep),
        ],
        out_specs=pl.BlockSpec((BLK, O), lambda i: (i, 0)),
        out_shape=jax.ShapeDtypeStruct((B, O), jnp.float32),
    )(pooled_t, coverage_t, w1e, w1f, w1c, b1r, W2, b2r)


def kernel(input_idx, obs_mask, coverage, emb_table, W1, b1, W2, b2):
    B, L = input_idx.shape
    ED = emb_table.shape[1]
    emb_pad = jnp.pad(emb_table, ((0, 0), (0, 1))).reshape(-1)
    pooled_t = _sc_pool(input_idx.reshape(-1), obs_mask.reshape(-1),
                        emb_pad, B, L)
    w1e = W1[:ED]
    w1f = W1[ED:ED + 1]
    w1c = W1[ED + 1:ED + 2]
    return _mlp(pooled_t, coverage.reshape(1, B),
                w1e, w1f, w1c, b1.reshape(1, -1), W2, b2.reshape(1, -1))
